# Initial kernel scaffold; baseline (speedup 1.0000x reference)
import jax
import jax.numpy as jnp
from jax.experimental import pallas as pl


def kernel(x, edge_index):
    raise NotImplementedError("write your pallas kernel here")


# speedup vs baseline: 3.0354x; 3.0354x over previous
import functools

import jax
import jax.numpy as jnp
from jax import lax
from jax.experimental import pallas as pl
from jax.experimental.pallas import tpu as pltpu
from jax.experimental.pallas import tpu_sc as plsc

_NC = 2
_NS = 16
_NW = _NC * _NS
_LANES = 16

_CHUNK = 8000


@functools.lru_cache(maxsize=None)
def _make_kernel(N, D, E):
    assert D % _NW == 0
    fw = D // _NW
    seg = fw * N
    assert seg % _LANES == 0 and seg % 8 == 0
    assert E % _CHUNK == 0 and _CHUNK % _LANES == 0
    n_chunks = E // _CHUNK
    n_blk = _CHUNK // _LANES

    mesh = plsc.VectorSubcoreMesh(core_axis_name="c", subcore_axis_name="s")

    @functools.partial(
        pl.kernel,
        out_type=jax.ShapeDtypeStruct((D * N,), jnp.float32),
        mesh=mesh,
        compiler_params=pltpu.CompilerParams(
            needs_layout_passes=False,
            use_tc_tiling_on_sc=False,
        ),
        scratch_types=[
            pltpu.VMEM((seg,), jnp.float32),
            pltpu.VMEM((seg,), jnp.float32),
            pltpu.VMEM((_CHUNK,), jnp.int32),
            pltpu.VMEM((_CHUNK,), jnp.int32),
        ],
    )
    def scatter_add_kernel(xt_hbm, src_hbm, dst_hbm, out_hbm,
                           x_v, acc_v, src_v, dst_v):
        cid = lax.axis_index("c")
        sid = lax.axis_index("s")
        wid = sid * _NC + cid
        base = wid * seg

        pltpu.sync_copy(xt_hbm.at[pl.ds(base, seg)], x_v)

        @pl.loop(0, seg // _LANES)
        def _zero(i):
            acc_v[pl.ds(i * _LANES, _LANES)] = jnp.zeros((_LANES,), jnp.float32)

        @pl.loop(0, n_chunks)
        def _per_chunk(ci):
            eoff = ci * _CHUNK
            pltpu.sync_copy(src_hbm.at[pl.ds(eoff, _CHUNK)], src_v)
            pltpu.sync_copy(dst_hbm.at[pl.ds(eoff, _CHUNK)], dst_v)

            @pl.loop(0, n_blk)
            def _per_block(bi):
                off = bi * _LANES
                s16 = src_v[pl.ds(off, _LANES)]
                d16 = dst_v[pl.ds(off, _LANES)]
                for f in range(fw):
                    g = plsc.load_gather(x_v, [s16 + f * N])
                    plsc.addupdate_scatter(acc_v, [d16 + f * N], g)

        pltpu.sync_copy(acc_v, out_hbm.at[pl.ds(base, seg)])

    return scatter_add_kernel


def kernel(x, edge_index):
    N, D = x.shape
    E = edge_index.shape[1]
    src = edge_index[0].astype(jnp.int32)
    dst = edge_index[1].astype(jnp.int32)
    xt = jnp.transpose(x).reshape(-1)
    out_flat = _make_kernel(N, D, E)(xt, src, dst)
    return jnp.transpose(out_flat.reshape(D, N))


# speedup vs baseline: 3.5050x; 1.1547x over previous
import functools

import jax
import jax.numpy as jnp
from jax import lax
from jax.experimental import pallas as pl
from jax.experimental.pallas import tpu as pltpu
from jax.experimental.pallas import tpu_sc as plsc

_NC = 2
_NS = 16
_NW = _NC * _NS
_LANES = 16

_CHUNK = 8000


@functools.lru_cache(maxsize=None)
def _make_kernel(N, D, E):
    assert D % _NW == 0
    fw = D // _NW
    seg = fw * N
    assert seg % _LANES == 0 and seg % 8 == 0
    assert E % _CHUNK == 0 and _CHUNK % _LANES == 0
    n_chunks = E // _CHUNK
    n_blk = _CHUNK // _LANES

    mesh = plsc.VectorSubcoreMesh(core_axis_name="c", subcore_axis_name="s")

    @functools.partial(
        pl.kernel,
        out_type=jax.ShapeDtypeStruct((D * N,), jnp.float32),
        mesh=mesh,
        compiler_params=pltpu.CompilerParams(
            needs_layout_passes=False,
            use_tc_tiling_on_sc=False,
        ),
        scratch_types=[
            pltpu.VMEM((seg,), jnp.float32),
            pltpu.VMEM((seg,), jnp.float32),
            pltpu.VMEM((2, _CHUNK), jnp.int32),
            pltpu.VMEM((2, _CHUNK), jnp.int32),
            pltpu.SemaphoreType.DMA,
            pltpu.SemaphoreType.DMA,
        ],
    )
    def scatter_add_kernel(xt_hbm, src_hbm, dst_hbm, out_hbm,
                           x_v, acc_v, src_v, dst_v, sem0, sem1):
        cid = lax.axis_index("c")
        sid = lax.axis_index("s")
        wid = sid * _NC + cid
        base = wid * seg
        sems = (sem0, sem1)

        def start_fetch(ci, b):
            eoff = lax.min(ci, n_chunks - 1) * _CHUNK
            pltpu.async_copy(src_hbm.at[pl.ds(eoff, _CHUNK)], src_v.at[b], sems[b])
            pltpu.async_copy(dst_hbm.at[pl.ds(eoff, _CHUNK)], dst_v.at[b], sems[b])

        def wait_fetch(b):
            pltpu.make_async_copy(src_hbm.at[pl.ds(0, _CHUNK)], src_v.at[b], sems[b]).wait()
            pltpu.make_async_copy(dst_hbm.at[pl.ds(0, _CHUNK)], dst_v.at[b], sems[b]).wait()

        start_fetch(0, 0)
        pltpu.sync_copy(xt_hbm.at[pl.ds(base, seg)], x_v)

        @pl.loop(0, seg // _LANES, unroll=8)
        def _zero(i):
            acc_v[pl.ds(i * _LANES, _LANES)] = jnp.zeros((_LANES,), jnp.float32)

        def process(b):
            @pl.loop(0, n_blk, unroll=8)
            def _per_block(bi):
                off = bi * _LANES
                s16 = src_v[b, pl.ds(off, _LANES)]
                d16 = dst_v[b, pl.ds(off, _LANES)]
                for f in range(fw):
                    g = plsc.load_gather(x_v, [s16 + f * N])
                    plsc.addupdate_scatter(acc_v, [d16 + f * N], g)

        @pl.loop(0, n_chunks // 2)
        def _per_pair(pi):
            ci = pi * 2
            start_fetch(ci + 1, 1)
            wait_fetch(0)
            process(0)
            start_fetch(ci + 2, 0)
            wait_fetch(1)
            process(1)

        wait_fetch(0)
        pltpu.sync_copy(acc_v, out_hbm.at[pl.ds(base, seg)])

    return scatter_add_kernel


def kernel(x, edge_index):
    N, D = x.shape
    E = edge_index.shape[1]
    src = edge_index[0].astype(jnp.int32)
    dst = edge_index[1].astype(jnp.int32)
    xt = jnp.transpose(x).reshape(-1)
    out_flat = _make_kernel(N, D, E)(xt, src, dst)
    return jnp.transpose(out_flat.reshape(D, N))


# speedup vs baseline: 10.1405x; 2.8932x over previous
import functools

import jax
import jax.numpy as jnp
from jax import lax
from jax.experimental import pallas as pl
from jax.experimental.pallas import tpu as pltpu
from jax.experimental.pallas import tpu_sc as plsc

_NC = 2
_NS = 16
_LANES = 16
_K = 125


@functools.lru_cache(maxsize=None)
def _make_kernel(N, D, E):
    assert D % _NC == 0
    dh = D // _NC
    assert dh % _LANES == 0
    rpt = N // _NS
    ept = E // _NS
    assert N % _NS == 0 and E % _NS == 0
    assert ept % _K == 0 and rpt % _K == 0
    n_ops = ept // _K
    assert n_ops % 2 == 0
    n_zero = rpt // _K

    mesh = plsc.VectorSubcoreMesh(core_axis_name="c", subcore_axis_name="s")

    @functools.partial(
        pl.kernel,
        out_type=jax.ShapeDtypeStruct((_NC, N, dh), jnp.float32),
        mesh=mesh,
        compiler_params=pltpu.CompilerParams(
            needs_layout_passes=False,
            use_tc_tiling_on_sc=False,
        ),
        scratch_types=[
            pltpu.VMEM((n_ops, _K), jnp.int32),
            pltpu.VMEM((n_ops, _K), jnp.int32),
            pltpu.VMEM((2, _K, dh), jnp.float32),
            pltpu.VMEM((_K, dh), jnp.float32),
            pltpu.VMEM_SHARED((N, dh), jnp.float32),
            pltpu.SemaphoreType.DMA,
            pltpu.SemaphoreType.DMA,
        ],
    )
    def scatter_add_kernel(xs_hbm, src_hbm, dst_hbm, out_hbm,
                           src_v, dst_v, rows_v, zero_v, acc_sh, g0, g1):
        cid = lax.axis_index("c")
        sid = lax.axis_index("s")
        gsems = (g0, g1)
        table = xs_hbm.at[cid]

        pltpu.sync_copy(src_hbm.at[sid], src_v)
        pltpu.sync_copy(dst_hbm.at[sid], dst_v)

        @pl.loop(0, _K)
        def _zero_row(r):
            for j in range(dh // _LANES):
                zero_v[r, pl.ds(j * _LANES, _LANES)] = (
                    jnp.zeros((_LANES,), jnp.float32))

        @pl.loop(0, n_zero)
        def _zero_acc(r):
            pltpu.sync_copy(
                zero_v, acc_sh.at[pl.ds(sid * rpt + r * _K, _K)])

        plsc.subcore_barrier()

        def start_gather(j, b):
            jj = lax.min(j, n_ops - 1)
            pltpu.async_copy(table.at[src_v.at[jj]], rows_v.at[b], gsems[b])

        def wait_gather(b):
            pltpu.make_async_copy(
                table.at[src_v.at[0]], rows_v.at[b], gsems[b]).wait()

        def scatter_add(j, b):
            pltpu.sync_copy(rows_v.at[b], acc_sh.at[dst_v.at[j]], add=True)

        start_gather(0, 0)

        @pl.loop(0, n_ops // 2)
        def _per_pair(pi):
            j0 = pi * 2
            start_gather(j0 + 1, 1)
            wait_gather(0)
            scatter_add(j0, 0)
            start_gather(j0 + 2, 0)
            wait_gather(1)
            scatter_add(j0 + 1, 1)

        wait_gather(0)

        plsc.subcore_barrier()
        pltpu.sync_copy(acc_sh.at[pl.ds(sid * rpt, rpt)],
                        out_hbm.at[cid, pl.ds(sid * rpt, rpt)])

    return scatter_add_kernel


def kernel(x, edge_index):
    N, D = x.shape
    E = edge_index.shape[1]
    src = edge_index[0].astype(jnp.int32)
    dst = edge_index[1].astype(jnp.int32)
    dh = D // _NC
    ept = E // _NS
    xs = jnp.stack([x[:, :dh], x[:, dh:]])
    src_r = src.reshape(_NS, ept // _K, _K)
    dst_r = dst.reshape(_NS, ept // _K, _K)
    out = _make_kernel(N, D, E)(xs, src_r, dst_r)
    return jnp.concatenate([out[0], out[1]], axis=1)


# speedup vs baseline: 10.8752x; 1.0725x over previous
import functools

import jax
import jax.numpy as jnp
from jax import lax
from jax.experimental import pallas as pl
from jax.experimental.pallas import tpu as pltpu
from jax.experimental.pallas import tpu_sc as plsc

_NC = 2
_NS = 16
_LANES = 16
_K = 125


@functools.lru_cache(maxsize=None)
def _make_kernel(N, D, E):
    assert D % _NC == 0
    dh = D // _NC
    assert dh % _LANES == 0
    rpt = N // _NS
    ept = E // _NS
    assert N % _NS == 0 and E % _NS == 0
    assert ept % _K == 0 and rpt % _K == 0
    n_ops = ept // _K
    assert n_ops % 4 == 0
    n_zero = rpt // _K

    mesh = plsc.VectorSubcoreMesh(core_axis_name="c", subcore_axis_name="s")

    @functools.partial(
        pl.kernel,
        out_type=jax.ShapeDtypeStruct((_NC, N, dh), jnp.float32),
        mesh=mesh,
        compiler_params=pltpu.CompilerParams(
            needs_layout_passes=False,
            use_tc_tiling_on_sc=False,
        ),
        scratch_types=[
            pltpu.VMEM((n_ops, _K), jnp.int32),
            pltpu.VMEM((n_ops, _K), jnp.int32),
            pltpu.VMEM((4, _K, dh), jnp.float32),
            pltpu.VMEM((_K, dh), jnp.float32),
            pltpu.VMEM_SHARED((N, dh), jnp.float32),
            pltpu.SemaphoreType.DMA,
            pltpu.SemaphoreType.DMA,
            pltpu.SemaphoreType.DMA,
            pltpu.SemaphoreType.DMA,
            pltpu.SemaphoreType.DMA,
            pltpu.SemaphoreType.DMA,
            pltpu.SemaphoreType.DMA,
            pltpu.SemaphoreType.DMA,
        ],
    )
    def scatter_add_kernel(xs_hbm, src_hbm, dst_hbm, out_hbm,
                           src_v, dst_v, rows_v, zero_v, acc_sh,
                           g0, g1, g2, g3, s0, s1, s2, s3):
        cid = lax.axis_index("c")
        sid = lax.axis_index("s")
        gsems = (g0, g1, g2, g3)
        ssems = (s0, s1, s2, s3)
        table = xs_hbm.at[cid]

        pltpu.sync_copy(src_hbm.at[sid], src_v)
        pltpu.sync_copy(dst_hbm.at[sid], dst_v)

        @pl.loop(0, _K)
        def _zero_row(r):
            for j in range(dh // _LANES):
                zero_v[r, pl.ds(j * _LANES, _LANES)] = (
                    jnp.zeros((_LANES,), jnp.float32))

        @pl.loop(0, n_zero)
        def _zero_acc(r):
            pltpu.sync_copy(
                zero_v, acc_sh.at[pl.ds(sid * rpt + r * _K, _K)])

        plsc.subcore_barrier()

        def start_gather(j, b):
            jj = lax.min(j, n_ops - 1)
            pltpu.async_copy(table.at[src_v.at[jj]], rows_v.at[b], gsems[b])

        def wait_gather(b):
            pltpu.make_async_copy(
                table.at[src_v.at[0]], rows_v.at[b], gsems[b]).wait()

        def start_scatter(j, b):
            pltpu.async_copy(
                rows_v.at[b], acc_sh.at[dst_v.at[j]], ssems[b], add=True)

        def wait_scatter(b):
            pltpu.make_async_copy(
                rows_v.at[b], acc_sh.at[dst_v.at[0]], ssems[b]).wait()

        for b in range(4):
            start_gather(b, b)

        @pl.loop(0, n_ops // 4)
        def _per_group(gi):
            j0 = gi * 4
            for b in range(4):
                wait_gather(b)
                start_scatter(j0 + b, b)
            for b in range(4):
                wait_scatter(b)
                start_gather(j0 + 4 + b, b)

        for b in range(4):
            wait_gather(b)

        plsc.subcore_barrier()
        pltpu.sync_copy(acc_sh.at[pl.ds(sid * rpt, rpt)],
                        out_hbm.at[cid, pl.ds(sid * rpt, rpt)])

    return scatter_add_kernel


def kernel(x, edge_index):
    N, D = x.shape
    E = edge_index.shape[1]
    src = edge_index[0].astype(jnp.int32)
    dst = edge_index[1].astype(jnp.int32)
    dh = D // _NC
    ept = E // _NS
    xs = jnp.stack([x[:, :dh], x[:, dh:]])
    src_r = src.reshape(_NS, ept // _K, _K)
    dst_r = dst.reshape(_NS, ept // _K, _K)
    out = _make_kernel(N, D, E)(xs, src_r, dst_r)
    return jnp.concatenate([out[0], out[1]], axis=1)


# speedup vs baseline: 11.9519x; 1.0990x over previous
import functools

import jax
import jax.numpy as jnp
from jax import lax
from jax.experimental import pallas as pl
from jax.experimental.pallas import tpu as pltpu
from jax.experimental.pallas import tpu_sc as plsc

_NC = 2
_NS = 16
_LANES = 16
_K = 125


@functools.lru_cache(maxsize=None)
def _make_kernel(N, D, E):
    assert D % _NC == 0
    dh = D // _NC
    assert dh % _LANES == 0
    rpt = N // _NS
    ept = E // _NS
    assert N % _NS == 0 and E % _NS == 0
    assert ept % _K == 0 and rpt % _K == 0
    n_ops = ept // _K
    assert n_ops % 4 == 0
    n_zero = rpt // _K

    mesh = plsc.VectorSubcoreMesh(core_axis_name="c", subcore_axis_name="s")

    @functools.partial(
        pl.kernel,
        out_type=jax.ShapeDtypeStruct((N, D), jnp.float32),
        mesh=mesh,
        compiler_params=pltpu.CompilerParams(
            needs_layout_passes=False,
            use_tc_tiling_on_sc=False,
        ),
        scratch_types=[
            pltpu.VMEM((n_ops, _K), jnp.int32),
            pltpu.VMEM((n_ops, _K), jnp.int32),
            pltpu.VMEM((4, _K, dh), jnp.float32),
            pltpu.VMEM((_K, dh), jnp.float32),
            pltpu.VMEM_SHARED((N, dh), jnp.float32),
            pltpu.SemaphoreType.DMA,
            pltpu.SemaphoreType.DMA,
            pltpu.SemaphoreType.DMA,
            pltpu.SemaphoreType.DMA,
            pltpu.SemaphoreType.DMA,
            pltpu.SemaphoreType.DMA,
            pltpu.SemaphoreType.DMA,
            pltpu.SemaphoreType.DMA,
        ],
    )
    def scatter_add_kernel(xs_hbm, src_hbm, dst_hbm, out_hbm,
                           src_v, dst_v, rows_v, zero_v, acc_sh,
                           g0, g1, g2, g3, s0, s1, s2, s3):
        cid = lax.axis_index("c")
        sid = lax.axis_index("s")
        gsems = (g0, g1, g2, g3)
        ssems = (s0, s1, s2, s3)
        table = xs_hbm.at[cid]

        pltpu.sync_copy(src_hbm.at[sid], src_v)
        pltpu.sync_copy(dst_hbm.at[sid], dst_v)

        @pl.loop(0, _K)
        def _zero_row(r):
            for j in range(dh // _LANES):
                zero_v[r, pl.ds(j * _LANES, _LANES)] = (
                    jnp.zeros((_LANES,), jnp.float32))

        @pl.loop(0, n_zero)
        def _zero_acc(r):
            pltpu.sync_copy(
                zero_v, acc_sh.at[pl.ds(sid * rpt + r * _K, _K)])

        plsc.subcore_barrier()

        def start_gather(j, b):
            jj = lax.min(j, n_ops - 1)
            pltpu.async_copy(table.at[src_v.at[jj]], rows_v.at[b], gsems[b])

        def wait_gather(b):
            pltpu.make_async_copy(
                table.at[src_v.at[0]], rows_v.at[b], gsems[b]).wait()

        def start_scatter(j, b):
            pltpu.async_copy(
                rows_v.at[b], acc_sh.at[dst_v.at[j]], ssems[b], add=True)

        def wait_scatter(b):
            pltpu.make_async_copy(
                rows_v.at[b], acc_sh.at[dst_v.at[0]], ssems[b]).wait()

        for b in range(4):
            start_gather(b, b)

        @pl.loop(0, n_ops // 4)
        def _per_group(gi):
            j0 = gi * 4
            for b in range(4):
                wait_gather(b)
                start_scatter(j0 + b, b)
            for b in range(4):
                wait_scatter(b)
                start_gather(j0 + 4 + b, b)

        for b in range(4):
            wait_gather(b)

        plsc.subcore_barrier()
        pltpu.sync_copy(acc_sh.at[pl.ds(sid * rpt, rpt)],
                        out_hbm.at[pl.ds(sid * rpt, rpt),
                                   pl.ds(cid * dh, dh)])

    return scatter_add_kernel


def kernel(x, edge_index):
    N, D = x.shape
    E = edge_index.shape[1]
    src = edge_index[0].astype(jnp.int32)
    dst = edge_index[1].astype(jnp.int32)
    dh = D // _NC
    ept = E // _NS
    xs = jnp.stack([x[:, :dh], x[:, dh:]])
    src_r = src.reshape(_NS, ept // _K, _K)
    dst_r = dst.reshape(_NS, ept // _K, _K)
    return _make_kernel(N, D, E)(xs, src_r, dst_r)


# speedup vs baseline: 13.0242x; 1.0897x over previous
import functools

import jax
import jax.numpy as jnp
from jax import lax
from jax.experimental import pallas as pl
from jax.experimental.pallas import tpu as pltpu
from jax.experimental.pallas import tpu_sc as plsc

_NC = 2
_NS = 16
_LANES = 16
_K = 125


@functools.lru_cache(maxsize=None)
def _make_kernel(N, D, E):
    assert D % _NC == 0
    dh = D // _NC
    assert dh % _LANES == 0
    rpt = N // _NS
    ept = E // _NS
    assert N % _NS == 0 and E % _NS == 0
    assert ept % _K == 0 and rpt % _K == 0
    n_ops = ept // _K
    assert n_ops % 4 == 0
    n_zero = rpt // _K

    mesh = plsc.VectorSubcoreMesh(core_axis_name="c", subcore_axis_name="s")

    @functools.partial(
        pl.kernel,
        out_type=jax.ShapeDtypeStruct((N, D), jnp.float32),
        mesh=mesh,
        compiler_params=pltpu.CompilerParams(
            needs_layout_passes=False,
            use_tc_tiling_on_sc=False,
        ),
        scratch_types=[
            pltpu.VMEM((n_ops, _K), jnp.int32),
            pltpu.VMEM((n_ops, _K), jnp.int32),
            pltpu.VMEM((4, _K, dh), jnp.float32),
            pltpu.VMEM((_K, dh), jnp.float32),
            pltpu.VMEM_SHARED((N, dh), jnp.float32),
            pltpu.SemaphoreType.DMA,
            pltpu.SemaphoreType.DMA,
            pltpu.SemaphoreType.DMA,
            pltpu.SemaphoreType.DMA,
            pltpu.SemaphoreType.DMA,
            pltpu.SemaphoreType.DMA,
            pltpu.SemaphoreType.DMA,
            pltpu.SemaphoreType.DMA,
        ],
    )
    def scatter_add_kernel(xs_hbm, src_hbm, dst_hbm, out_hbm,
                           src_v, dst_v, rows_v, zero_v, acc_sh,
                           g0, g1, g2, g3, s0, s1, s2, s3):
        cid = lax.axis_index("c")
        sid = lax.axis_index("s")
        gsems = (g0, g1, g2, g3)
        ssems = (s0, s1, s2, s3)
        table = xs_hbm

        pltpu.sync_copy(src_hbm.at[cid, sid], src_v)
        pltpu.sync_copy(dst_hbm.at[sid], dst_v)

        @pl.loop(0, _K)
        def _zero_row(r):
            for j in range(dh // _LANES):
                zero_v[r, pl.ds(j * _LANES, _LANES)] = (
                    jnp.zeros((_LANES,), jnp.float32))

        @pl.loop(0, n_zero)
        def _zero_acc(r):
            pltpu.sync_copy(
                zero_v, acc_sh.at[pl.ds(sid * rpt + r * _K, _K)])

        plsc.subcore_barrier()

        def start_gather(j, b):
            jj = lax.min(j, n_ops - 1)
            pltpu.async_copy(table.at[src_v.at[jj]], rows_v.at[b], gsems[b])

        def wait_gather(b):
            pltpu.make_async_copy(
                table.at[src_v.at[0]], rows_v.at[b], gsems[b]).wait()

        def start_scatter(j, b):
            pltpu.async_copy(
                rows_v.at[b], acc_sh.at[dst_v.at[j]], ssems[b], add=True)

        def wait_scatter(b):
            pltpu.make_async_copy(
                rows_v.at[b], acc_sh.at[dst_v.at[0]], ssems[b]).wait()

        for b in range(4):
            start_gather(b, b)

        @pl.loop(0, n_ops // 4)
        def _per_group(gi):
            j0 = gi * 4
            for b in range(4):
                wait_gather(b)
                start_scatter(j0 + b, b)
            for b in range(4):
                wait_scatter(b)
                start_gather(j0 + 4 + b, b)

        for b in range(4):
            wait_gather(b)

        plsc.subcore_barrier()
        pltpu.sync_copy(acc_sh.at[pl.ds(sid * rpt, rpt)],
                        out_hbm.at[pl.ds(sid * rpt, rpt),
                                   pl.ds(cid * dh, dh)])

    return scatter_add_kernel


def kernel(x, edge_index):
    N, D = x.shape
    E = edge_index.shape[1]
    src = edge_index[0].astype(jnp.int32)
    dst = edge_index[1].astype(jnp.int32)
    dh = D // _NC
    ept = E // _NS
    xs = x.reshape(_NC * N, dh)
    src2 = src * 2
    src_r = jnp.stack([src2, src2 + 1]).reshape(_NC, _NS, ept // _K, _K)
    dst_r = dst.reshape(_NS, ept // _K, _K)
    return _make_kernel(N, D, E)(xs, src_r, dst_r)
